# baseline (speedup 1.0000x reference)
import functools

import jax
import jax.numpy as jnp
from jax import lax
from jax.experimental import pallas as pl
from jax.experimental.pallas import tpu as pltpu
from jax.experimental.pallas import tpu_sc as plsc

_HIDDEN = 64
_BATCH = 16384
_NITEMS = 1000000
_NC, _NS, _L = 2, 16, 16
_NW = _NC * _NS
_BPW = _BATCH // _NW
_NCHUNK = 4
_CHUNK = _BPW // _NCHUNK

_BLK = 16384
_GRID = (_NITEMS + _BLK - 1) // _BLK


def _project_body(w_ref, tabT_ref, out_ref):
    out_ref[...] = jnp.dot(w_ref[...], tabT_ref[...],
                           preferred_element_type=jnp.float32)[0]


_tc_project = pl.pallas_call(
    _project_body,
    grid=(_GRID,),
    in_specs=[
        pl.BlockSpec((1, _HIDDEN), lambda j: (0, 0)),
        pl.BlockSpec((_HIDDEN, _BLK), lambda j: (0, j)),
    ],
    out_specs=pl.BlockSpec((_BLK,), lambda j: (j,)),
    out_shape=jax.ShapeDtypeStruct((_NITEMS,), jnp.float32),
)

_mesh = plsc.VectorSubcoreMesh(core_axis_name="c", subcore_axis_name="s")


@functools.partial(
    pl.kernel,
    mesh=_mesh,
    out_type=jax.ShapeDtypeStruct((_BATCH,), jnp.float32),
    scratch_types=[
        pltpu.VMEM((_NCHUNK, _CHUNK), jnp.int32),
        pltpu.VMEM((_BPW,), jnp.float32),
        pltpu.VMEM((_L,), jnp.float32),
        pltpu.SemaphoreType.DMA,
    ],
)
def _sc_gather_bias(ids_hbm, proj_hbm, b_hbm, out_hbm,
                    idx_v, vals_v, b_v, sem):
    wid = lax.axis_index("s") * _NC + lax.axis_index("c")

    pltpu.sync_copy(b_hbm, b_v)
    pltpu.sync_copy(ids_hbm.at[pl.ds(wid * _NCHUNK, _NCHUNK)], idx_v)

    copies = [
        pltpu.async_copy(proj_hbm.at[idx_v.at[j]],
                         vals_v.at[pl.ds(j * _CHUNK, _CHUNK)], sem)
        for j in range(_NCHUNK)
    ]
    for c in copies:
        c.wait()

    bias_vec = b_v[...]
    for v in range(_BPW // _L):
        vals_v[pl.ds(v * _L, _L)] = vals_v[pl.ds(v * _L, _L)] + bias_vec

    pltpu.sync_copy(vals_v, out_hbm.at[pl.ds(wid * _BPW, _BPW)])


def kernel(item_ids, emb_table, fc_w, fc_b):
    ids2d = item_ids.astype(jnp.int32).reshape(_NW * _NCHUNK, _CHUNK)
    tabT = emb_table.T
    proj = _tc_project(fc_w.astype(jnp.float32), tabT)
    bias_vec = jnp.broadcast_to(fc_b.astype(jnp.float32), (_L,))
    out = _sc_gather_bias(ids2d, proj, bias_vec)
    return out.reshape(_BATCH, 1)


# speedup vs baseline: 1.1083x; 1.1083x over previous
import functools

import jax
import jax.numpy as jnp
from jax import lax
from jax.experimental import pallas as pl
from jax.experimental.pallas import tpu as pltpu
from jax.experimental.pallas import tpu_sc as plsc

_HIDDEN = 64
_BATCH = 16384
_NITEMS = 1000000
_NC, _NS, _L = 2, 16, 16
_NW = _NC * _NS
_BPW = _BATCH // _NW
_NCHUNK = 4
_CHUNK = _BPW // _NCHUNK

_BLK = 32768
_GRID = (_NITEMS + _BLK - 1) // _BLK


def _project_body(w_ref, tabT_ref, out_ref):
    out_ref[...] = jnp.dot(w_ref[...], tabT_ref[...],
                           preferred_element_type=jnp.float32)[0]


_tc_project = pl.pallas_call(
    _project_body,
    grid=(_GRID,),
    in_specs=[
        pl.BlockSpec((1, _HIDDEN), lambda j: (0, 0)),
        pl.BlockSpec((_HIDDEN, _BLK), lambda j: (0, j)),
    ],
    out_specs=pl.BlockSpec((_BLK,), lambda j: (j,)),
    out_shape=jax.ShapeDtypeStruct((_NITEMS,), jnp.float32),
)

_mesh = plsc.VectorSubcoreMesh(core_axis_name="c", subcore_axis_name="s")


@functools.partial(
    pl.kernel,
    mesh=_mesh,
    out_type=jax.ShapeDtypeStruct((_BATCH,), jnp.float32),
    scratch_types=[
        pltpu.VMEM((_NCHUNK, _CHUNK), jnp.int32),
        pltpu.VMEM((_BPW,), jnp.float32),
        pltpu.VMEM((_L,), jnp.float32),
        pltpu.SemaphoreType.DMA,
    ],
)
def _sc_gather_bias(ids_hbm, proj_hbm, b_hbm, out_hbm,
                    idx_v, vals_v, b_v, sem):
    wid = lax.axis_index("s") * _NC + lax.axis_index("c")

    pltpu.sync_copy(b_hbm, b_v)
    pltpu.sync_copy(ids_hbm.at[pl.ds(wid * _NCHUNK, _NCHUNK)], idx_v)

    copies = [
        pltpu.async_copy(proj_hbm.at[idx_v.at[j]],
                         vals_v.at[pl.ds(j * _CHUNK, _CHUNK)], sem)
        for j in range(_NCHUNK)
    ]
    for c in copies:
        c.wait()

    bias_vec = b_v[...]
    for v in range(_BPW // _L):
        vals_v[pl.ds(v * _L, _L)] = vals_v[pl.ds(v * _L, _L)] + bias_vec

    pltpu.sync_copy(vals_v, out_hbm.at[pl.ds(wid * _BPW, _BPW)])


def kernel(item_ids, emb_table, fc_w, fc_b):
    ids2d = item_ids.astype(jnp.int32).reshape(_NW * _NCHUNK, _CHUNK)
    tabT = emb_table.T
    proj = _tc_project(fc_w.astype(jnp.float32), tabT)
    bias_vec = jnp.broadcast_to(fc_b.astype(jnp.float32), (_L,))
    out = _sc_gather_bias(ids2d, proj, bias_vec)
    return out.reshape(_BATCH, 1)


# speedup vs baseline: 1.1330x; 1.0223x over previous
import functools

import jax
import jax.numpy as jnp
from jax import lax
from jax.experimental import pallas as pl
from jax.experimental.pallas import tpu as pltpu
from jax.experimental.pallas import tpu_sc as plsc

_HIDDEN = 64
_BATCH = 16384
_NITEMS = 1000000
_NC, _NS, _L = 2, 16, 16
_NW = _NC * _NS
_BPW = _BATCH // _NW
_NCHUNK = 4
_CHUNK = _BPW // _NCHUNK

_BLK = 32768
_GRID = (_NITEMS + _BLK - 1) // _BLK


def _project_body(w_ref, b_ref, tabT_ref, out_ref):
    out_ref[...] = jnp.dot(w_ref[...], tabT_ref[...],
                           preferred_element_type=jnp.float32)[0] + b_ref[0]


_tc_project = pl.pallas_call(
    _project_body,
    grid=(_GRID,),
    in_specs=[
        pl.BlockSpec((1, _HIDDEN), lambda j: (0, 0)),
        pl.BlockSpec(memory_space=pltpu.SMEM),
        pl.BlockSpec((_HIDDEN, _BLK), lambda j: (0, j)),
    ],
    out_specs=pl.BlockSpec((_BLK,), lambda j: (j,)),
    out_shape=jax.ShapeDtypeStruct((_NITEMS,), jnp.float32),
)

_mesh = plsc.VectorSubcoreMesh(core_axis_name="c", subcore_axis_name="s")


@functools.partial(
    pl.kernel,
    mesh=_mesh,
    out_type=jax.ShapeDtypeStruct((_BATCH,), jnp.float32),
    scratch_types=[
        pltpu.VMEM((_NCHUNK, _CHUNK), jnp.int32),
        pltpu.VMEM((_BPW,), jnp.float32),
        pltpu.SemaphoreType.DMA,
    ],
)
def _sc_gather(ids_hbm, proj_hbm, out_hbm, idx_v, vals_v, sem):
    wid = lax.axis_index("s") * _NC + lax.axis_index("c")

    pltpu.sync_copy(ids_hbm.at[pl.ds(wid * _NCHUNK, _NCHUNK)], idx_v)

    copies = [
        pltpu.async_copy(proj_hbm.at[idx_v.at[j]],
                         vals_v.at[pl.ds(j * _CHUNK, _CHUNK)], sem)
        for j in range(_NCHUNK)
    ]
    for c in copies:
        c.wait()

    pltpu.sync_copy(vals_v, out_hbm.at[pl.ds(wid * _BPW, _BPW)])


def kernel(item_ids, emb_table, fc_w, fc_b):
    ids2d = item_ids.astype(jnp.int32).reshape(_NW * _NCHUNK, _CHUNK)
    tabT = emb_table.T
    proj = _tc_project(fc_w.astype(jnp.float32),
                       fc_b.astype(jnp.float32), tabT)
    out = _sc_gather(ids2d, proj)
    return out.reshape(_BATCH, 1)
